# Initial kernel scaffold; baseline (speedup 1.0000x reference)
import jax
import jax.numpy as jnp
from jax.experimental import pallas as pl


def kernel(x, pos_table):
    raise NotImplementedError("write your pallas kernel here")


# speedup vs baseline: 1.7293x; 1.7293x over previous
import jax
import jax.numpy as jnp
from jax.experimental import pallas as pl

_MAXLEN = 8192
_EMBED = 1024
_BATCH = 4
_BS = 512


def _add_kernel(x_ref, pos_ref, out_ref):
    out_ref[...] = x_ref[...] + pos_ref[...][None, :, :]


def kernel(x, pos_table):
    x = jnp.reshape(x, (-1, _MAXLEN, _EMBED))
    grid = (_MAXLEN // _BS,)
    return pl.pallas_call(
        _add_kernel,
        grid=grid,
        in_specs=[
            pl.BlockSpec((_BATCH, _BS, _EMBED), lambda i: (0, i, 0)),
            pl.BlockSpec((_BS, _EMBED), lambda i: (i, 0)),
        ],
        out_specs=pl.BlockSpec((_BATCH, _BS, _EMBED), lambda i: (0, i, 0)),
        out_shape=jax.ShapeDtypeStruct((_BATCH, _MAXLEN, _EMBED), x.dtype),
    )(x, pos_table)
